# baseline (speedup 1.0000x reference)
import functools

import jax
import jax.numpy as jnp
from jax import lax
from jax.experimental import pallas as pl
from jax.experimental.pallas import tpu as pltpu
from jax.experimental.pallas import tpu_sc as plsc

NC = 2
NS = 16
NW = NC * NS
LANES = 16
NBUF = 4
LOOKAHEAD = 2


def _sc_add(x, pe):
    seq, d = x.shape
    rows_per_w = seq // NW
    chunk = min(rows_per_w, max(1, 8192 // d))
    n_chunks = rows_per_w // chunk

    mesh = plsc.VectorSubcoreMesh(core_axis_name="c", subcore_axis_name="s")

    @functools.partial(
        pl.kernel,
        out_type=jax.ShapeDtypeStruct((seq, d), jnp.float32),
        mesh=mesh,
        scratch_types=[
            pltpu.VMEM((NBUF, chunk, d), jnp.float32),
            pltpu.VMEM((NBUF, chunk), jnp.int32),
            pltpu.SemaphoreType.DMA((NBUF,)),
            pltpu.SemaphoreType.DMA((NBUF,)),
            pltpu.SemaphoreType.DMA((NBUF,)),
        ],
    )
    def k(x_hbm, p_hbm, o_hbm, x_v, idx_v, ldx_sem, ldp_sem, st_sem):
        wid = lax.axis_index("s") * NC + lax.axis_index("c")
        base = wid * rows_per_w

        xloads = {}
        gathers = {}
        stores = {}

        def start_x(c):
            b = c % NBUF
            off = base + c * chunk

            @plsc.parallel_loop(0, chunk // LANES, unroll=2)
            def _(i):
                idx_v[b, pl.ds(i * LANES, LANES)] = (
                    lax.broadcasted_iota(jnp.int32, (LANES,), 0)
                    + (off + i * LANES)
                )

            xloads[c] = pltpu.make_async_copy(
                x_hbm.at[pl.ds(off, chunk), :], x_v.at[b], ldx_sem.at[b]
            )
            xloads[c].start()

        def start_gather_add(c):
            b = c % NBUF
            gathers[c] = pltpu.async_copy(
                p_hbm.at[idx_v.at[b]], x_v.at[b], ldp_sem.at[b], add=True
            )

        def start_store(c):
            b = c % NBUF
            off = base + c * chunk
            stores[c] = pltpu.make_async_copy(
                x_v.at[b], o_hbm.at[pl.ds(off, chunk), :], st_sem.at[b]
            )
            stores[c].start()

        for c in range(min(LOOKAHEAD, n_chunks)):
            start_x(c)

        for c in range(n_chunks):
            xloads.pop(c).wait()
            start_gather_add(c)
            if c >= 1:
                gathers.pop(c - 1).wait()
                start_store(c - 1)
            nxt = c + LOOKAHEAD
            if nxt < n_chunks:
                prev = nxt - NBUF
                if prev >= 0:
                    stores.pop(prev).wait()
                start_x(nxt)

        gathers.pop(n_chunks - 1).wait()
        start_store(n_chunks - 1)
        for dsc in stores.values():
            dsc.wait()

    return k(x, pe)


@jax.jit
def _combined(x, pe):
    s, d = x.shape
    w = s * d // 128
    out = _sc_add(x.reshape(w, 128), pe[:s].reshape(w, 128))
    return out.reshape(s, d)


def kernel(x, pe):
    return _combined(x, pe)


# speedup vs baseline: 1.2788x; 1.2788x over previous
import functools

import jax
import jax.numpy as jnp
from jax import lax
from jax.experimental import pallas as pl
from jax.experimental.pallas import tpu as pltpu
from jax.experimental.pallas import tpu_sc as plsc

NC = 2
NS = 16
NW = NC * NS
LANES = 16
NBUF = 4


@jax.jit
def _sc_add(x, pe):
    seq, d = x.shape
    rows_per_w = seq // NW
    chunk = min(rows_per_w, max(1, 8192 // d))
    n_chunks = rows_per_w // chunk
    vecs_per_row = d // LANES

    mesh = plsc.VectorSubcoreMesh(core_axis_name="c", subcore_axis_name="s")

    @functools.partial(
        pl.kernel,
        out_type=jax.ShapeDtypeStruct((seq, d), jnp.float32),
        mesh=mesh,
        scratch_types=[
            pltpu.VMEM((NBUF, chunk, d), jnp.float32),
            pltpu.VMEM((NBUF, chunk, d), jnp.float32),
            pltpu.SemaphoreType.DMA((NBUF,)),
            pltpu.SemaphoreType.DMA((NBUF,)),
            pltpu.SemaphoreType.DMA((NBUF,)),
        ],
    )
    def k(x_hbm, p_hbm, o_hbm, x_v, p_v, ldx_sem, ldp_sem, st_sem):
        wid = lax.axis_index("s") * NC + lax.axis_index("c")
        base = wid * rows_per_w

        loads = {}
        stores = {}

        def start_load(c):
            b = c % NBUF
            off = base + c * chunk
            loads[c] = (
                pltpu.make_async_copy(
                    x_hbm.at[pl.ds(off, chunk), :], x_v.at[b], ldx_sem.at[b]
                ),
                pltpu.make_async_copy(
                    p_hbm.at[pl.ds(off, chunk), :], p_v.at[b], ldp_sem.at[b]
                ),
            )
            loads[c][0].start()
            loads[c][1].start()

        for c in range(min(NBUF, n_chunks)):
            start_load(c)

        for c in range(n_chunks):
            b = c % NBUF
            for dsc in loads.pop(c):
                dsc.wait()

            @plsc.parallel_loop(0, chunk, unroll=8)
            def _(r):
                for j in range(vecs_per_row):
                    s = pl.ds(j * LANES, LANES)
                    x_v[b, r, s] = x_v[b, r, s] + p_v[b, r, s]

            off = base + c * chunk
            stores[c] = pltpu.make_async_copy(
                x_v.at[b], o_hbm.at[pl.ds(off, chunk), :], st_sem.at[b]
            )
            stores[c].start()

            nxt = c + NBUF
            if nxt < n_chunks:
                stores.pop(nxt - NBUF).wait()
                start_load(nxt)

        for dsc in stores.values():
            dsc.wait()

    return k(x, pe)


def kernel(x, pe):
    return _sc_add(x, pe[: x.shape[0]])


# speedup vs baseline: 1.2871x; 1.0065x over previous
import functools

import jax
import jax.numpy as jnp
from jax import lax
from jax.experimental import pallas as pl
from jax.experimental.pallas import tpu as pltpu
from jax.experimental.pallas import tpu_sc as plsc

NC = 2
NS = 16
NW = NC * NS
LANES = 16
NBUF = 4


@jax.jit
def _sc_add(x, pe):
    seq, d = x.shape
    rows_per_w = seq // NW
    chunk = min(rows_per_w, max(1, 8192 // d))
    n_chunks = rows_per_w // chunk
    vecs_per_row = d // LANES

    mesh = plsc.VectorSubcoreMesh(core_axis_name="c", subcore_axis_name="s")

    @functools.partial(
        pl.kernel,
        out_type=jax.ShapeDtypeStruct((seq, d), jnp.float32),
        mesh=mesh,
        compiler_params=pltpu.CompilerParams(allow_input_fusion=[True, True]),
        scratch_types=[
            pltpu.VMEM((NBUF, chunk, d), jnp.float32),
            pltpu.VMEM((NBUF, chunk, d), jnp.float32),
            pltpu.SemaphoreType.DMA((NBUF,)),
            pltpu.SemaphoreType.DMA((NBUF,)),
            pltpu.SemaphoreType.DMA((NBUF,)),
        ],
    )
    def k(x_hbm, p_hbm, o_hbm, x_v, p_v, ldx_sem, ldp_sem, st_sem):
        wid = lax.axis_index("s") * NC + lax.axis_index("c")
        base = wid * rows_per_w

        loads = {}
        stores = {}

        def start_load(c):
            b = c % NBUF
            off = base + c * chunk
            loads[c] = (
                pltpu.make_async_copy(
                    x_hbm.at[pl.ds(off, chunk), :], x_v.at[b], ldx_sem.at[b]
                ),
                pltpu.make_async_copy(
                    p_hbm.at[pl.ds(off, chunk), :], p_v.at[b], ldp_sem.at[b]
                ),
            )
            loads[c][0].start()
            loads[c][1].start()

        for c in range(min(NBUF, n_chunks)):
            start_load(c)

        for c in range(n_chunks):
            b = c % NBUF
            for dsc in loads.pop(c):
                dsc.wait()

            @plsc.parallel_loop(0, chunk, unroll=8)
            def _(r):
                for j in range(vecs_per_row):
                    s = pl.ds(j * LANES, LANES)
                    x_v[b, r, s] = x_v[b, r, s] + p_v[b, r, s]

            off = base + c * chunk
            stores[c] = pltpu.make_async_copy(
                x_v.at[b], o_hbm.at[pl.ds(off, chunk), :], st_sem.at[b]
            )
            stores[c].start()

            nxt = c + NBUF
            if nxt < n_chunks:
                stores.pop(nxt - NBUF).wait()
                start_load(nxt)

        for dsc in stores.values():
            dsc.wait()

    return k(x, pe)


def kernel(x, pe):
    return _sc_add(x, pe[: x.shape[0]])
